# Initial kernel scaffold; baseline (speedup 1.0000x reference)
import jax
import jax.numpy as jnp
from jax.experimental import pallas as pl


def kernel(segment_ids, feats, W_fc, b_fc, W_gate, b_gate, W_pool, b_pool):
    raise NotImplementedError("write your pallas kernel here")


# speedup vs baseline: 4.1440x; 4.1440x over previous
import functools

import jax
import jax.numpy as jnp
from jax.experimental import pallas as pl
from jax.experimental.pallas import tpu as pltpu

G = 64


def _fused_kernel(seg_ref, feats_ref, wfc_ref, bfc_ref, wg_ref, bg_ref,
                  wp_ref, bp_ref, classes_ref, pred_ref,
                  m_ref, s_ref, r_ref, *, nblocks, bm, n):
    i = pl.program_id(0)

    @pl.when(i == 0)
    def _init():
        m_ref[...] = jnp.full((1, G), -jnp.inf, dtype=jnp.float32)
        s_ref[...] = jnp.zeros((1, G), dtype=jnp.float32)
        r_ref[...] = jnp.zeros_like(r_ref)

    f = feats_ref[...]
    row = jax.lax.broadcasted_iota(jnp.int32, (bm, 1), 0) + i * bm
    valid = row < n
    f = jnp.where(valid, f, 0.0)

    cls = jax.lax.dot_general(
        f, wfc_ref[...], (((1,), (1,)), ((), ())),
        preferred_element_type=jnp.float32,
        precision=jax.lax.Precision.HIGHEST)
    classes_ref[...] = cls + bfc_ref[...]

    g = jnp.sum(f * wg_ref[...], axis=1, keepdims=True) + bg_ref[0, 0]

    seg = seg_ref[0]
    seg_col = seg.reshape(bm, 1)
    lane = jax.lax.broadcasted_iota(jnp.int32, (bm, G), 1)
    oh = (seg_col == lane) & valid

    neg_inf = jnp.float32(-jnp.inf)
    bmax = jnp.max(jnp.where(oh, g, neg_inf), axis=0, keepdims=True)
    m_old = m_ref[...]
    m_new = jnp.maximum(m_old, bmax)
    alpha = jnp.where(m_old == neg_inf, 0.0, jnp.exp(m_old - m_new))
    e = jnp.exp(jnp.where(oh, g - m_new, neg_inf))

    m_ref[...] = m_new
    s_ref[...] = s_ref[...] * alpha + jnp.sum(e, axis=0, keepdims=True)
    contrib = jax.lax.dot_general(
        e, f, (((0,), (0,)), ((), ())),
        preferred_element_type=jnp.float32,
        precision=jax.lax.Precision.HIGHEST)
    r_ref[...] = r_ref[...] * alpha.reshape(G, 1) + contrib

    @pl.when(i == nblocks - 1)
    def _finish():
        denom = s_ref[...].reshape(G, 1) + 1e-12
        readout = r_ref[...] / denom
        pred = jax.lax.dot_general(
            readout, wp_ref[...], (((1,), (1,)), ((), ())),
            preferred_element_type=jnp.float32,
            precision=jax.lax.Precision.HIGHEST)
        pred_ref[...] = pred + bp_ref[...]


@jax.jit
def kernel(segment_ids, feats, W_fc, b_fc, W_gate, b_gate, W_pool, b_pool):
    n, d = feats.shape
    c = W_fc.shape[0]
    bm = 1024
    nblocks = pl.cdiv(n, bm)
    npad = nblocks * bm

    seg = segment_ids.astype(jnp.int32)
    seg = jnp.concatenate(
        [seg, jnp.full((npad - n,), jnp.int32(2 ** 30))]) if npad > n else seg
    seg3 = seg.reshape(nblocks, 1, bm)

    grid_spec = pltpu.PrefetchScalarGridSpec(
        num_scalar_prefetch=0,
        grid=(nblocks,),
        in_specs=[
            pl.BlockSpec((1, 1, bm), lambda i: (i, 0, 0)),
            pl.BlockSpec((bm, d), lambda i: (i, 0)),
            pl.BlockSpec((c, d), lambda i: (0, 0)),
            pl.BlockSpec((1, c), lambda i: (0, 0)),
            pl.BlockSpec((1, d), lambda i: (0, 0)),
            pl.BlockSpec((1, 1), lambda i: (0, 0)),
            pl.BlockSpec((c, d), lambda i: (0, 0)),
            pl.BlockSpec((1, c), lambda i: (0, 0)),
        ],
        out_specs=[
            pl.BlockSpec((bm, c), lambda i: (i, 0)),
            pl.BlockSpec((G, c), lambda i: (0, 0)),
        ],
        scratch_shapes=[
            pltpu.VMEM((1, G), jnp.float32),
            pltpu.VMEM((1, G), jnp.float32),
            pltpu.VMEM((G, d), jnp.float32),
        ],
    )

    classes, pred = pl.pallas_call(
        functools.partial(_fused_kernel, nblocks=nblocks, bm=bm, n=n),
        grid_spec=grid_spec,
        out_shape=[
            jax.ShapeDtypeStruct((n, c), jnp.float32),
            jax.ShapeDtypeStruct((G, c), jnp.float32),
        ],
        compiler_params=pltpu.CompilerParams(
            dimension_semantics=("arbitrary",),
        ),
    )(seg3, feats, W_fc, b_fc.reshape(1, c), W_gate, b_gate.reshape(1, 1),
      W_pool, b_pool.reshape(1, c))
    return (classes, pred)


# speedup vs baseline: 10.2353x; 2.4699x over previous
import functools

import jax
import jax.numpy as jnp
from jax.experimental import pallas as pl
from jax.experimental.pallas import tpu as pltpu

G = 64


def _fused_kernel(seg_ref, feats_ref, wfc_ref, bfc_ref, wg_ref, bg_ref,
                  wp_ref, bp_ref, classes_ref, pred_ref,
                  m_ref, s_ref, r_ref, *, nblocks, bm, n):
    i = pl.program_id(0)

    @pl.when(i == 0)
    def _init():
        m_ref[...] = jnp.full((1, G), -jnp.inf, dtype=jnp.float32)
        s_ref[...] = jnp.zeros((1, G), dtype=jnp.float32)
        r_ref[...] = jnp.zeros_like(r_ref)

    f = feats_ref[...]
    row = jax.lax.broadcasted_iota(jnp.int32, (bm, 1), 0) + i * bm
    valid = row < n
    f = jnp.where(valid, f, 0.0)

    cls = jax.lax.dot_general(
        f, wfc_ref[...], (((1,), (1,)), ((), ())),
        preferred_element_type=jnp.float32,
        precision=jax.lax.Precision.DEFAULT)
    classes_ref[...] = cls + bfc_ref[...]

    g = jnp.sum(f * wg_ref[...], axis=1, keepdims=True) + bg_ref[0, 0]

    seg = seg_ref[0]
    seg_col = seg.reshape(bm, 1)
    lane = jax.lax.broadcasted_iota(jnp.int32, (bm, G), 1)
    oh = (seg_col == lane) & valid

    neg_inf = jnp.float32(-jnp.inf)
    bmax = jnp.max(jnp.where(oh, g, neg_inf), axis=0, keepdims=True)
    m_old = m_ref[...]
    m_new = jnp.maximum(m_old, bmax)
    alpha = jnp.where(m_old == neg_inf, 0.0, jnp.exp(m_old - m_new))
    e = jnp.exp(jnp.where(oh, g - m_new, neg_inf))

    m_ref[...] = m_new
    s_ref[...] = s_ref[...] * alpha + jnp.sum(e, axis=0, keepdims=True)
    contrib = jax.lax.dot_general(
        e, f, (((0,), (0,)), ((), ())),
        preferred_element_type=jnp.float32,
        precision=jax.lax.Precision.DEFAULT)
    r_ref[...] = r_ref[...] * alpha.reshape(G, 1) + contrib

    @pl.when(i == nblocks - 1)
    def _finish():
        denom = s_ref[...].reshape(G, 1) + 1e-12
        readout = r_ref[...] / denom
        pred = jax.lax.dot_general(
            readout, wp_ref[...], (((1,), (1,)), ((), ())),
            preferred_element_type=jnp.float32,
            precision=jax.lax.Precision.DEFAULT)
        pred_ref[...] = pred + bp_ref[...]


@jax.jit
def kernel(segment_ids, feats, W_fc, b_fc, W_gate, b_gate, W_pool, b_pool):
    n, d = feats.shape
    c = W_fc.shape[0]
    bm = 1024
    nblocks = pl.cdiv(n, bm)
    npad = nblocks * bm

    seg = segment_ids.astype(jnp.int32)
    seg = jnp.concatenate(
        [seg, jnp.full((npad - n,), jnp.int32(2 ** 30))]) if npad > n else seg
    seg3 = seg.reshape(nblocks, 1, bm)

    grid_spec = pltpu.PrefetchScalarGridSpec(
        num_scalar_prefetch=0,
        grid=(nblocks,),
        in_specs=[
            pl.BlockSpec((1, 1, bm), lambda i: (i, 0, 0)),
            pl.BlockSpec((bm, d), lambda i: (i, 0)),
            pl.BlockSpec((c, d), lambda i: (0, 0)),
            pl.BlockSpec((1, c), lambda i: (0, 0)),
            pl.BlockSpec((1, d), lambda i: (0, 0)),
            pl.BlockSpec((1, 1), lambda i: (0, 0)),
            pl.BlockSpec((c, d), lambda i: (0, 0)),
            pl.BlockSpec((1, c), lambda i: (0, 0)),
        ],
        out_specs=[
            pl.BlockSpec((bm, c), lambda i: (i, 0)),
            pl.BlockSpec((G, c), lambda i: (0, 0)),
        ],
        scratch_shapes=[
            pltpu.VMEM((1, G), jnp.float32),
            pltpu.VMEM((1, G), jnp.float32),
            pltpu.VMEM((G, d), jnp.float32),
        ],
    )

    classes, pred = pl.pallas_call(
        functools.partial(_fused_kernel, nblocks=nblocks, bm=bm, n=n),
        grid_spec=grid_spec,
        out_shape=[
            jax.ShapeDtypeStruct((n, c), jnp.float32),
            jax.ShapeDtypeStruct((G, c), jnp.float32),
        ],
        compiler_params=pltpu.CompilerParams(
            dimension_semantics=("arbitrary",),
        ),
    )(seg3, feats, W_fc, b_fc.reshape(1, c), W_gate, b_gate.reshape(1, 1),
      W_pool, b_pool.reshape(1, c))
    return (classes, pred)


# speedup vs baseline: 11.9851x; 1.1710x over previous
import functools

import jax
import jax.numpy as jnp
from jax.experimental import pallas as pl
from jax.experimental.pallas import tpu as pltpu

G = 64


def _fused_kernel(seg_ref, feats_ref, wfc_ref, bfc_ref, wg_ref, bg_ref,
                  wp_ref, bp_ref, classes_ref, pred_ref,
                  m_ref, s_ref, r_ref, *, nblocks, bm):
    i = pl.program_id(0)

    @pl.when(i == 0)
    def _init():
        m_ref[...] = jnp.full((1, G), -jnp.inf, dtype=jnp.float32)
        s_ref[...] = jnp.zeros((1, G), dtype=jnp.float32)
        r_ref[...] = jnp.zeros_like(r_ref)

    f = feats_ref[...]
    fb = f.astype(jnp.bfloat16)

    cls = jax.lax.dot_general(
        fb, wfc_ref[...], (((1,), (1,)), ((), ())),
        preferred_element_type=jnp.float32)
    classes_ref[...] = cls + bfc_ref[...]

    g = jnp.sum(f * wg_ref[...], axis=1, keepdims=True) + bg_ref[0, 0]

    seg = seg_ref[0]
    seg_col = seg.reshape(bm, 1)
    lane = jax.lax.broadcasted_iota(jnp.int32, (bm, G), 1)
    oh = seg_col == lane

    neg_inf = jnp.float32(-jnp.inf)
    bmax = jnp.max(jnp.where(oh, g, neg_inf), axis=0, keepdims=True)
    m_old = m_ref[...]
    m_new = jnp.maximum(m_old, bmax)
    alpha = jnp.where(m_old == neg_inf, 0.0, jnp.exp(m_old - m_new))
    e = jnp.exp(jnp.where(oh, g - m_new, neg_inf))

    m_ref[...] = m_new
    s_ref[...] = s_ref[...] * alpha + jnp.sum(e, axis=0, keepdims=True)
    contrib = jax.lax.dot_general(
        fb, e.astype(jnp.bfloat16), (((0,), (0,)), ((), ())),
        preferred_element_type=jnp.float32)
    r_ref[...] = r_ref[...] * alpha + contrib

    @pl.when(i == nblocks - 1)
    def _finish():
        readout = r_ref[...] / (s_ref[...] + 1e-12)
        pred = jax.lax.dot_general(
            readout, wp_ref[...], (((0,), (1,)), ((), ())),
            preferred_element_type=jnp.float32,
            precision=jax.lax.Precision.HIGHEST)
        pred_ref[...] = pred + bp_ref[...]


@jax.jit
def kernel(segment_ids, feats, W_fc, b_fc, W_gate, b_gate, W_pool, b_pool):
    n, d = feats.shape
    c = W_fc.shape[0]
    bm = 2000 if n % 2000 == 0 else n
    nblocks = n // bm
    seg3 = segment_ids.astype(jnp.int32).reshape(nblocks, 1, bm)

    grid_spec = pltpu.PrefetchScalarGridSpec(
        num_scalar_prefetch=0,
        grid=(nblocks,),
        in_specs=[
            pl.BlockSpec((1, 1, bm), lambda i: (i, 0, 0)),
            pl.BlockSpec((bm, d), lambda i: (i, 0)),
            pl.BlockSpec((c, d), lambda i: (0, 0)),
            pl.BlockSpec((1, c), lambda i: (0, 0)),
            pl.BlockSpec((1, d), lambda i: (0, 0)),
            pl.BlockSpec((1, 1), lambda i: (0, 0)),
            pl.BlockSpec((c, d), lambda i: (0, 0)),
            pl.BlockSpec((1, c), lambda i: (0, 0)),
        ],
        out_specs=[
            pl.BlockSpec((bm, c), lambda i: (i, 0)),
            pl.BlockSpec((G, c), lambda i: (0, 0)),
        ],
        scratch_shapes=[
            pltpu.VMEM((1, G), jnp.float32),
            pltpu.VMEM((1, G), jnp.float32),
            pltpu.VMEM((d, G), jnp.float32),
        ],
    )

    classes, pred = pl.pallas_call(
        functools.partial(_fused_kernel, nblocks=nblocks, bm=bm),
        grid_spec=grid_spec,
        out_shape=[
            jax.ShapeDtypeStruct((n, c), jnp.float32),
            jax.ShapeDtypeStruct((G, c), jnp.float32),
        ],
        compiler_params=pltpu.CompilerParams(
            dimension_semantics=("arbitrary",),
        ),
    )(seg3, feats, W_fc.astype(jnp.bfloat16), b_fc.reshape(1, c), W_gate,
      b_gate.reshape(1, 1), W_pool, b_pool.reshape(1, c))
    return (classes, pred)
